# baseline (speedup 1.0000x reference)
import functools
import jax
import jax.numpy as jnp
from jax import lax
from jax.experimental import pallas as pl
from jax.experimental.pallas import tpu as pltpu
from jax.experimental.pallas import tpu_sc as plsc

_NC = 2
_NS = 16
_NW = _NC * _NS
_L = 16
_D = 64
_C = 128
_SCALE = 8.0


def _make_sc_lookup(nch):
    mesh = plsc.VectorSubcoreMesh(
        core_axis_name="c", subcore_axis_name="s",
        num_cores=_NC, num_subcores=_NS)

    @functools.partial(
        pl.kernel,
        mesh=mesh,
        out_type=jax.ShapeDtypeStruct((_NW, nch, _C, _D), jnp.float32),
        scratch_types=[
            pltpu.VMEM((nch, _C), jnp.int32),
            pltpu.VMEM((_C, _D), jnp.float32),
            pltpu.VMEM((_C, _D), jnp.float32),
            pltpu.SemaphoreType.DMA,
        ],
        compiler_params=pltpu.CompilerParams(use_tc_tiling_on_sc=False),
    )
    def sc_lookup(x_hbm, table_hbm, out_hbm, idx_v, gbuf, sbuf, gsem):
        wid = lax.axis_index("s") * _NC + lax.axis_index("c")
        pltpu.sync_copy(x_hbm.at[wid], idx_v)

        def chunk(j, carry):
            pltpu.async_copy(table_hbm.at[idx_v.at[j]], gbuf, gsem).wait()

            def row(r, c2):
                for c4 in range(_D // _L):
                    sbuf[r, pl.ds(c4 * _L, _L)] = (
                        gbuf[r, pl.ds(c4 * _L, _L)] * _SCALE)
                return c2

            lax.fori_loop(0, _C, row, 0)
            pltpu.sync_copy(sbuf, out_hbm.at[wid, j])
            return carry

        lax.fori_loop(0, nch, chunk, 0)

    return sc_lookup


def kernel(x, table):
    b, s = x.shape
    n = b * s
    assert n % (_NW * _C) == 0
    nch = n // (_NW * _C)
    xw = x.reshape(_NW, nch, _C).astype(jnp.int32)
    out = _make_sc_lookup(nch)(xw, table)
    return out.reshape(b, s, _D)


# speedup vs baseline: 1.1875x; 1.1875x over previous
import functools
import jax
import jax.numpy as jnp
from jax import lax
from jax.experimental import pallas as pl
from jax.experimental.pallas import tpu as pltpu
from jax.experimental.pallas import tpu_sc as plsc

_NC = 2
_NS = 16
_NW = _NC * _NS
_L = 16
_D = 64
_C = 128
_NBUF = 4
_SCALE = 8.0


def _make_sc_lookup(nch):
    assert nch % _NBUF == 0
    mesh = plsc.VectorSubcoreMesh(
        core_axis_name="c", subcore_axis_name="s",
        num_cores=_NC, num_subcores=_NS)

    @functools.partial(
        pl.kernel,
        mesh=mesh,
        out_type=jax.ShapeDtypeStruct((_NW, nch, _C, _D), jnp.float32),
        scratch_types=[
            pltpu.VMEM((nch, _C), jnp.int32),
            pltpu.VMEM((_NBUF, _C, _D), jnp.float32),
            pltpu.VMEM((_NBUF, _C, _D), jnp.float32),
            pltpu.SemaphoreType.DMA,
            pltpu.SemaphoreType.DMA,
        ],
        compiler_params=pltpu.CompilerParams(use_tc_tiling_on_sc=False),
    )
    def sc_lookup(x_hbm, table_hbm, out_hbm, idx_v, gbuf, sbuf, gsem, ssem):
        wid = lax.axis_index("s") * _NC + lax.axis_index("c")
        pltpu.sync_copy(x_hbm.at[wid], idx_v)

        for b in range(_NBUF):
            pltpu.async_copy(table_hbm.at[idx_v.at[b]], gbuf.at[b], gsem)

        def group(g, carry):
            for b in range(_NBUF):
                j = g * _NBUF + b
                pltpu.make_async_copy(
                    table_hbm.at[idx_v.at[j]], gbuf.at[b], gsem).wait()
                @pl.when(g > 0)
                def _():
                    pltpu.make_async_copy(
                        sbuf.at[b], out_hbm.at[wid, j], ssem).wait()

                @plsc.parallel_loop(0, _C, 1, unroll=4)
                def _(r):
                    for c4 in range(_D // _L):
                        sbuf[b, r, pl.ds(c4 * _L, _L)] = (
                            gbuf[b, r, pl.ds(c4 * _L, _L)] * _SCALE)

                pltpu.async_copy(sbuf.at[b], out_hbm.at[wid, j], ssem)
                @pl.when(j + _NBUF < nch)
                def _():
                    pltpu.async_copy(
                        table_hbm.at[idx_v.at[j + _NBUF]], gbuf.at[b], gsem)
            return carry

        lax.fori_loop(0, nch // _NBUF, group, 0)
        for b in range(_NBUF):
            pltpu.make_async_copy(
                sbuf.at[b], out_hbm.at[wid, nch - _NBUF + b], ssem).wait()

    return sc_lookup


def kernel(x, table):
    b, s = x.shape
    n = b * s
    assert n % (_NW * _C) == 0
    nch = n // (_NW * _C)
    xw = x.reshape(_NW, nch, _C).astype(jnp.int32)
    out = _make_sc_lookup(nch)(xw, table)
    return out.reshape(b, s, _D)
